# baseline (speedup 1.0000x reference)
import functools

import jax
import jax.numpy as jnp
from jax import lax
from jax.experimental import pallas as pl
from jax.experimental.pallas import tpu as pltpu
from jax.experimental.pallas import tpu_sc as plsc

N = 50000
D = 512
G = 256
B = 2000
NB = N // B

NW = 32
C = 1568
NPAD = NW * C


def _stats_body(batch_ref, x_ref, q_ref, wk_ref, sc_ref, m_out, s_out,
                m_ref, s_ref):
    i = pl.program_id(0)
    neg = jnp.float32(-jnp.inf)

    @pl.when(i == 0)
    def _init():
        m_ref[...] = jnp.full((G, 1), neg, jnp.float32)
        s_ref[...] = jnp.zeros((G, 1), jnp.float32)

    x = x_ref[...]
    b = batch_ref[0]
    k = jax.lax.dot_general(
        x, wk_ref[...], (((1,), (1,)), ((), ())),
        preferred_element_type=jnp.float32)
    scores = jax.lax.dot_general(
        q_ref[...], k, (((1,), (1,)), ((), ())),
        preferred_element_type=jnp.float32)
    sc_ref[0] = scores

    gids = jax.lax.broadcasted_iota(jnp.int32, (G, B), 0)
    oh = gids == b
    m_blk = jnp.max(jnp.where(oh, scores, neg), axis=1, keepdims=True)
    m_old = m_ref[...]
    m_new = jnp.maximum(m_old, m_blk)
    scale = jnp.where(m_new == neg, 0.0, jnp.exp(m_old - m_new))
    m_row = jnp.max(jnp.where(oh, m_new, neg), axis=0, keepdims=True)
    e = jnp.exp(scores - m_row)
    w = jnp.where(oh, e, 0.0)
    s_ref[...] = s_ref[...] * scale + jnp.sum(w, axis=1, keepdims=True)
    m_ref[...] = m_new

    @pl.when(i == NB - 1)
    def _fin():
        mf = m_ref[...]
        m_out[...] = jnp.where(mf == neg, 0.0, mf)
        s_out[...] = s_ref[...]


def _stage1(b3, x, q2, Wk):
    return pl.pallas_call(
        _stats_body,
        grid=(NB,),
        in_specs=[
            pl.BlockSpec((1, 1, B), lambda i: (i, 0, 0)),
            pl.BlockSpec((B, D), lambda i: (i, 0)),
            pl.BlockSpec((1, D), lambda i: (0, 0)),
            pl.BlockSpec((D, D), lambda i: (0, 0)),
        ],
        out_specs=[pl.BlockSpec((1, 1, B), lambda i: (i, 0, 0)),
                   pl.BlockSpec((G, 1), lambda i: (0, 0)),
                   pl.BlockSpec((G, 1), lambda i: (0, 0))],
        out_shape=[jax.ShapeDtypeStruct((NB, 1, B), jnp.float32),
                   jax.ShapeDtypeStruct((G, 1), jnp.float32),
                   jax.ShapeDtypeStruct((G, 1), jnp.float32)],
        scratch_shapes=[pltpu.VMEM((G, 1), jnp.float32),
                        pltpu.VMEM((G, 1), jnp.float32)],
    )(b3, x, q2, Wk)


def _sc_softmax_body(scores_hbm, batch_hbm, m_hbm, s_hbm, soft_hbm,
                     sco_v, b_v, soft_v, m_v, s_v):
    wid = lax.axis_index("s") * 2 + lax.axis_index("c")
    base = wid * C
    pltpu.sync_copy(scores_hbm.at[pl.ds(base, C)], sco_v)
    pltpu.sync_copy(batch_hbm.at[pl.ds(base, C)], b_v)
    pltpu.sync_copy(m_hbm, m_v)
    pltpu.sync_copy(s_hbm, s_v)

    def body(j, carry):
        sv = sco_v[pl.ds(j * 16, 16)]
        bv = b_v[pl.ds(j * 16, 16)]
        mg = plsc.load_gather(m_v, [bv])
        sg = plsc.load_gather(s_v, [bv])
        soft_v[pl.ds(j * 16, 16)] = jnp.exp(sv - mg) / (sg + 1e-16)
        return carry

    lax.fori_loop(0, C // 16, body, 0)
    pltpu.sync_copy(soft_v, soft_hbm.at[pl.ds(base, C)])


@functools.cache
def _sc_softmax_fn():
    return functools.partial(
        pl.kernel,
        out_type=jax.ShapeDtypeStruct((NPAD,), jnp.float32),
        mesh=plsc.VectorSubcoreMesh(core_axis_name="c",
                                    subcore_axis_name="s"),
        compiler_params=pltpu.CompilerParams(needs_layout_passes=False),
        scratch_types=[
            pltpu.VMEM((C,), jnp.float32),
            pltpu.VMEM((C,), jnp.int32),
            pltpu.VMEM((C,), jnp.float32),
            pltpu.VMEM((G,), jnp.float32),
            pltpu.VMEM((G,), jnp.float32),
        ],
    )(_sc_softmax_body)


def _pool_body(soft_ref, batch_ref, x_ref, wv_ref, out_ref, acc_ref):
    i = pl.program_id(0)

    @pl.when(i == 0)
    def _init():
        acc_ref[...] = jnp.zeros((G, D), jnp.float32)

    x = x_ref[...]
    b = batch_ref[0]
    soft = soft_ref[0]
    gids = jax.lax.broadcasted_iota(jnp.int32, (G, B), 0)
    w = jnp.where(gids == b, soft, 0.0)
    acc_ref[...] += jax.lax.dot_general(
        w.astype(jnp.bfloat16), x.astype(jnp.bfloat16),
        (((1,), (0,)), ((), ())), preferred_element_type=jnp.float32)

    @pl.when(i == NB - 1)
    def _fin():
        out_ref[...] = jax.lax.dot_general(
            acc_ref[...], wv_ref[...], (((1,), (1,)), ((), ())),
            preferred_element_type=jnp.float32,
            precision=jax.lax.Precision.HIGHEST)


def _stage3(soft3, b3, x, Wv):
    return pl.pallas_call(
        _pool_body,
        grid=(NB,),
        in_specs=[
            pl.BlockSpec((1, 1, B), lambda i: (i, 0, 0)),
            pl.BlockSpec((1, 1, B), lambda i: (i, 0, 0)),
            pl.BlockSpec((B, D), lambda i: (i, 0)),
            pl.BlockSpec((D, D), lambda i: (0, 0)),
        ],
        out_specs=pl.BlockSpec((G, D), lambda i: (0, 0)),
        out_shape=jax.ShapeDtypeStruct((G, D), jnp.float32),
        scratch_shapes=[pltpu.VMEM((G, D), jnp.float32)],
    )(soft3, b3, x, Wv)


def kernel(x, batch, query, Wk, Wv):
    b3 = batch.reshape(NB, 1, B)
    q2 = query.reshape(1, D)
    scores3, m2, s2 = _stage1(b3, x, q2, Wk)
    scores_pad = jnp.concatenate(
        [scores3.reshape(N), jnp.zeros((NPAD - N,), jnp.float32)])
    batch_pad = jnp.concatenate(
        [batch, jnp.zeros((NPAD - N,), jnp.int32)])
    soft_pad = _sc_softmax_fn()(scores_pad, batch_pad,
                                m2.reshape(G), s2.reshape(G))
    soft3 = soft_pad[:N].reshape(NB, 1, B)
    return _stage3(soft3, b3, x, Wv)


# speedup vs baseline: 1.0255x; 1.0255x over previous
import functools

import jax
import jax.numpy as jnp
from jax import lax
from jax.experimental import pallas as pl
from jax.experimental.pallas import tpu as pltpu
from jax.experimental.pallas import tpu_sc as plsc

N = 50000
D = 512
G = 256
B = 2000
NB = N // B

NW = 32
C = 1568
NPAD = NW * C


def _stats_body(batch_ref, x_ref, q_ref, wk_ref, sc_ref, m_out, s_out,
                xb_ref, m_ref, s_ref):
    i = pl.program_id(0)
    neg = jnp.float32(-jnp.inf)

    @pl.when(i == 0)
    def _init():
        m_ref[...] = jnp.full((G, 1), neg, jnp.float32)
        s_ref[...] = jnp.zeros((G, 1), jnp.float32)

    x = x_ref[...]
    xb_ref[...] = x.astype(jnp.bfloat16)
    b = batch_ref[0]
    k = jax.lax.dot_general(
        x, wk_ref[...], (((1,), (1,)), ((), ())),
        preferred_element_type=jnp.float32)
    scores = jax.lax.dot_general(
        q_ref[...], k, (((1,), (1,)), ((), ())),
        preferred_element_type=jnp.float32)
    sc_ref[0] = scores

    gids = jax.lax.broadcasted_iota(jnp.int32, (G, B), 0)
    oh = gids == b
    m_blk = jnp.max(jnp.where(oh, scores, neg), axis=1, keepdims=True)
    m_old = m_ref[...]
    m_new = jnp.maximum(m_old, m_blk)
    scale = jnp.where(m_new == neg, 0.0, jnp.exp(m_old - m_new))
    m_row = jnp.max(jnp.where(oh, m_new, neg), axis=0, keepdims=True)
    e = jnp.exp(scores - m_row)
    w = jnp.where(oh, e, 0.0)
    s_ref[...] = s_ref[...] * scale + jnp.sum(w, axis=1, keepdims=True)
    m_ref[...] = m_new

    @pl.when(i == NB - 1)
    def _fin():
        mf = m_ref[...]
        m_out[...] = jnp.where(mf == neg, 0.0, mf)
        s_out[...] = s_ref[...]


def _stage1(b3, x, q2, Wk):
    return pl.pallas_call(
        _stats_body,
        grid=(NB,),
        in_specs=[
            pl.BlockSpec((1, 1, B), lambda i: (i, 0, 0)),
            pl.BlockSpec((B, D), lambda i: (i, 0)),
            pl.BlockSpec((1, D), lambda i: (0, 0)),
            pl.BlockSpec((D, D), lambda i: (0, 0)),
        ],
        out_specs=[pl.BlockSpec((1, 1, B), lambda i: (i, 0, 0)),
                   pl.BlockSpec((G, 1), lambda i: (0, 0)),
                   pl.BlockSpec((G, 1), lambda i: (0, 0)),
                   pl.BlockSpec((B, D), lambda i: (i, 0))],
        out_shape=[jax.ShapeDtypeStruct((NB, 1, B), jnp.float32),
                   jax.ShapeDtypeStruct((G, 1), jnp.float32),
                   jax.ShapeDtypeStruct((G, 1), jnp.float32),
                   jax.ShapeDtypeStruct((N, D), jnp.bfloat16)],
        scratch_shapes=[pltpu.VMEM((G, 1), jnp.float32),
                        pltpu.VMEM((G, 1), jnp.float32)],
    )(b3, x, q2, Wk)


def _sc_softmax_body(scores_hbm, batch_hbm, m_hbm, s_hbm, soft_hbm,
                     sco_v, b_v, soft_v, m_v, s_v):
    wid = lax.axis_index("s") * 2 + lax.axis_index("c")
    base = wid * C
    pltpu.sync_copy(scores_hbm.at[pl.ds(base, C)], sco_v)
    pltpu.sync_copy(batch_hbm.at[pl.ds(base, C)], b_v)
    pltpu.sync_copy(m_hbm, m_v)
    pltpu.sync_copy(s_hbm, s_v)

    def body(j, carry):
        sv = sco_v[pl.ds(j * 16, 16)]
        bv = b_v[pl.ds(j * 16, 16)]
        mg = plsc.load_gather(m_v, [bv])
        sg = plsc.load_gather(s_v, [bv])
        soft_v[pl.ds(j * 16, 16)] = jnp.exp(sv - mg) / (sg + 1e-16)
        return carry

    lax.fori_loop(0, C // 16, body, 0)
    pltpu.sync_copy(soft_v, soft_hbm.at[pl.ds(base, C)])


@functools.cache
def _sc_softmax_fn():
    return functools.partial(
        pl.kernel,
        out_type=jax.ShapeDtypeStruct((NPAD,), jnp.float32),
        mesh=plsc.VectorSubcoreMesh(core_axis_name="c",
                                    subcore_axis_name="s"),
        compiler_params=pltpu.CompilerParams(needs_layout_passes=False),
        scratch_types=[
            pltpu.VMEM((C,), jnp.float32),
            pltpu.VMEM((C,), jnp.int32),
            pltpu.VMEM((C,), jnp.float32),
            pltpu.VMEM((G,), jnp.float32),
            pltpu.VMEM((G,), jnp.float32),
        ],
    )(_sc_softmax_body)


def _pool_body(soft_ref, batch_ref, x_ref, wv_ref, out_ref, acc_ref):
    i = pl.program_id(0)

    @pl.when(i == 0)
    def _init():
        acc_ref[...] = jnp.zeros((G, D), jnp.float32)

    xb = x_ref[...]
    b = batch_ref[0]
    soft = soft_ref[0]
    gids = jax.lax.broadcasted_iota(jnp.int32, (G, B), 0)
    w = jnp.where(gids == b, soft, 0.0)
    acc_ref[...] += jax.lax.dot_general(
        w.astype(jnp.bfloat16), xb,
        (((1,), (0,)), ((), ())), preferred_element_type=jnp.float32)

    @pl.when(i == NB - 1)
    def _fin():
        out_ref[...] = jax.lax.dot_general(
            acc_ref[...], wv_ref[...], (((1,), (1,)), ((), ())),
            preferred_element_type=jnp.float32,
            precision=jax.lax.Precision.HIGHEST)


def _stage3(soft3, b3, x, Wv):
    return pl.pallas_call(
        _pool_body,
        grid=(NB,),
        in_specs=[
            pl.BlockSpec((1, 1, B), lambda i: (i, 0, 0)),
            pl.BlockSpec((1, 1, B), lambda i: (i, 0, 0)),
            pl.BlockSpec((B, D), lambda i: (i, 0)),
            pl.BlockSpec((D, D), lambda i: (0, 0)),
        ],
        out_specs=pl.BlockSpec((G, D), lambda i: (0, 0)),
        out_shape=jax.ShapeDtypeStruct((G, D), jnp.float32),
        scratch_shapes=[pltpu.VMEM((G, D), jnp.float32)],
    )(soft3, b3, x, Wv)


def kernel(x, batch, query, Wk, Wv):
    b3 = batch.reshape(NB, 1, B)
    q2 = query.reshape(1, D)
    scores3, m2, s2, xb = _stage1(b3, x, q2, Wk)
    scores_pad = jnp.concatenate(
        [scores3.reshape(N), jnp.zeros((NPAD - N,), jnp.float32)])
    batch_pad = jnp.concatenate(
        [batch, jnp.zeros((NPAD - N,), jnp.int32)])
    soft_pad = _sc_softmax_fn()(scores_pad, batch_pad,
                                m2.reshape(G), s2.reshape(G))
    soft3 = soft_pad[:N].reshape(NB, 1, B)
    return _stage3(soft3, b3, xb, Wv)


# speedup vs baseline: 1.1647x; 1.1357x over previous
import functools

import jax
import jax.numpy as jnp
from jax import lax
from jax.experimental import pallas as pl
from jax.experimental.pallas import tpu as pltpu
from jax.experimental.pallas import tpu_sc as plsc

N = 50000
D = 512
G = 256
B = 5000
NB = N // B

NW = 32
C = 1568
NPAD = NW * C


def _stats_body(batch_ref, x_ref, q_ref, wk_ref, sc_ref, m_out, s_out,
                xb_ref, m_ref, s_ref):
    i = pl.program_id(0)
    neg = jnp.float32(-jnp.inf)

    @pl.when(i == 0)
    def _init():
        m_ref[...] = jnp.full((G, 1), neg, jnp.float32)
        s_ref[...] = jnp.zeros((G, 1), jnp.float32)

    x = x_ref[...]
    xb_ref[...] = x.astype(jnp.bfloat16)
    b = batch_ref[0]
    k = jax.lax.dot_general(
        x, wk_ref[...], (((1,), (1,)), ((), ())),
        preferred_element_type=jnp.float32)
    scores = jax.lax.dot_general(
        q_ref[...], k, (((1,), (1,)), ((), ())),
        preferred_element_type=jnp.float32)
    sc_ref[0] = scores

    gids = jax.lax.broadcasted_iota(jnp.int32, (G, B), 0)
    oh = gids == b
    m_blk = jnp.max(jnp.where(oh, scores, neg), axis=1, keepdims=True)
    m_old = m_ref[...]
    m_new = jnp.maximum(m_old, m_blk)
    scale = jnp.where(m_new == neg, 0.0, jnp.exp(m_old - m_new))
    m_row = jnp.max(jnp.where(oh, m_new, neg), axis=0, keepdims=True)
    e = jnp.exp(scores - m_row)
    w = jnp.where(oh, e, 0.0)
    s_ref[...] = s_ref[...] * scale + jnp.sum(w, axis=1, keepdims=True)
    m_ref[...] = m_new

    @pl.when(i == NB - 1)
    def _fin():
        mf = m_ref[...]
        m_out[...] = jnp.where(mf == neg, 0.0, mf)
        s_out[...] = s_ref[...]


def _stage1(b3, x, q2, Wk):
    return pl.pallas_call(
        _stats_body,
        grid=(NB,),
        in_specs=[
            pl.BlockSpec((1, 1, B), lambda i: (i, 0, 0)),
            pl.BlockSpec((B, D), lambda i: (i, 0)),
            pl.BlockSpec((1, D), lambda i: (0, 0)),
            pl.BlockSpec((D, D), lambda i: (0, 0)),
        ],
        out_specs=[pl.BlockSpec((1, 1, B), lambda i: (i, 0, 0)),
                   pl.BlockSpec((G, 1), lambda i: (0, 0)),
                   pl.BlockSpec((G, 1), lambda i: (0, 0)),
                   pl.BlockSpec((B, D), lambda i: (i, 0))],
        out_shape=[jax.ShapeDtypeStruct((NB, 1, B), jnp.float32),
                   jax.ShapeDtypeStruct((G, 1), jnp.float32),
                   jax.ShapeDtypeStruct((G, 1), jnp.float32),
                   jax.ShapeDtypeStruct((N, D), jnp.bfloat16)],
        scratch_shapes=[pltpu.VMEM((G, 1), jnp.float32),
                        pltpu.VMEM((G, 1), jnp.float32)],
    )(b3, x, q2, Wk)


def _sc_softmax_body(scores_hbm, batch_hbm, m_hbm, s_hbm, soft_hbm,
                     sco_v, b_v, soft_v, m_v, s_v):
    wid = lax.axis_index("s") * 2 + lax.axis_index("c")
    base = wid * C
    pltpu.sync_copy(scores_hbm.at[pl.ds(base, C)], sco_v)
    pltpu.sync_copy(batch_hbm.at[pl.ds(base, C)], b_v)
    pltpu.sync_copy(m_hbm, m_v)
    pltpu.sync_copy(s_hbm, s_v)

    def body(j, carry):
        sv = sco_v[pl.ds(j * 16, 16)]
        bv = b_v[pl.ds(j * 16, 16)]
        mg = plsc.load_gather(m_v, [bv])
        sg = plsc.load_gather(s_v, [bv])
        soft_v[pl.ds(j * 16, 16)] = jnp.exp(sv - mg) / (sg + 1e-16)
        return carry

    lax.fori_loop(0, C // 16, body, 0)
    pltpu.sync_copy(soft_v, soft_hbm.at[pl.ds(base, C)])


@functools.cache
def _sc_softmax_fn():
    return functools.partial(
        pl.kernel,
        out_type=jax.ShapeDtypeStruct((NPAD,), jnp.float32),
        mesh=plsc.VectorSubcoreMesh(core_axis_name="c",
                                    subcore_axis_name="s"),
        compiler_params=pltpu.CompilerParams(needs_layout_passes=False),
        scratch_types=[
            pltpu.VMEM((C,), jnp.float32),
            pltpu.VMEM((C,), jnp.int32),
            pltpu.VMEM((C,), jnp.float32),
            pltpu.VMEM((G,), jnp.float32),
            pltpu.VMEM((G,), jnp.float32),
        ],
    )(_sc_softmax_body)


def _pool_body(soft_ref, batch_ref, x_ref, wv_ref, out_ref, acc_ref):
    i = pl.program_id(0)

    @pl.when(i == 0)
    def _init():
        acc_ref[...] = jnp.zeros((G, D), jnp.float32)

    xb = x_ref[...]
    b = batch_ref[0]
    soft = soft_ref[0]
    gids = jax.lax.broadcasted_iota(jnp.int32, (G, B), 0)
    w = jnp.where(gids == b, soft, 0.0)
    acc_ref[...] += jax.lax.dot_general(
        w.astype(jnp.bfloat16), xb,
        (((1,), (0,)), ((), ())), preferred_element_type=jnp.float32)

    @pl.when(i == NB - 1)
    def _fin():
        out_ref[...] = jax.lax.dot_general(
            acc_ref[...], wv_ref[...], (((1,), (1,)), ((), ())),
            preferred_element_type=jnp.float32,
            precision=jax.lax.Precision.HIGHEST)


def _stage3(soft3, b3, x, Wv):
    return pl.pallas_call(
        _pool_body,
        grid=(NB,),
        in_specs=[
            pl.BlockSpec((1, 1, B), lambda i: (i, 0, 0)),
            pl.BlockSpec((1, 1, B), lambda i: (i, 0, 0)),
            pl.BlockSpec((B, D), lambda i: (i, 0)),
            pl.BlockSpec((D, D), lambda i: (0, 0)),
        ],
        out_specs=pl.BlockSpec((G, D), lambda i: (0, 0)),
        out_shape=jax.ShapeDtypeStruct((G, D), jnp.float32),
        scratch_shapes=[pltpu.VMEM((G, D), jnp.float32)],
    )(soft3, b3, x, Wv)


def kernel(x, batch, query, Wk, Wv):
    b3 = batch.reshape(NB, 1, B)
    q2 = query.reshape(1, D)
    scores3, m2, s2, xb = _stage1(b3, x, q2, Wk)
    scores_pad = jnp.concatenate(
        [scores3.reshape(N), jnp.zeros((NPAD - N,), jnp.float32)])
    batch_pad = jnp.concatenate(
        [batch, jnp.zeros((NPAD - N,), jnp.int32)])
    soft_pad = _sc_softmax_fn()(scores_pad, batch_pad,
                                m2.reshape(G), s2.reshape(G))
    soft3 = soft_pad[:N].reshape(NB, 1, B)
    return _stage3(soft3, b3, xb, Wv)


# speedup vs baseline: 1.7220x; 1.4786x over previous
import jax
import jax.numpy as jnp
from jax.experimental import pallas as pl
from jax.experimental.pallas import tpu as pltpu

N = 50000
D = 512
G = 256
B = 5000
NB = N // B


def _body(batch_ref, x_ref, q_ref, wk_ref, wv_ref, out_ref,
          m_ref, s_ref, acc_ref):
    i = pl.program_id(0)
    neg = jnp.float32(-jnp.inf)

    @pl.when(i == 0)
    def _init():
        m_ref[...] = jnp.full((G, 1), neg, jnp.float32)
        s_ref[...] = jnp.zeros((G, 1), jnp.float32)
        acc_ref[...] = jnp.zeros((G, D), jnp.float32)

    x = x_ref[...]
    b = batch_ref[0]
    k = jax.lax.dot_general(
        x, wk_ref[...], (((1,), (1,)), ((), ())),
        preferred_element_type=jnp.float32)
    scores = jax.lax.dot_general(
        q_ref[...], k, (((1,), (1,)), ((), ())),
        preferred_element_type=jnp.float32)
    gids = jax.lax.broadcasted_iota(jnp.int32, (G, B), 0)
    oh = gids == b

    m_blk = jnp.max(jnp.where(oh, scores, neg), axis=1, keepdims=True)
    m_old = m_ref[...]
    m_new = jnp.maximum(m_old, m_blk)
    scale = jnp.where(m_new == neg, 0.0, jnp.exp(m_old - m_new))
    m_row = jnp.max(jnp.where(oh, m_new, neg), axis=0, keepdims=True)
    e = jnp.exp(scores - m_row)
    w = jnp.where(oh, e, 0.0)
    s_ref[...] = s_ref[...] * scale + jnp.sum(w, axis=1, keepdims=True)
    acc_ref[...] = acc_ref[...] * scale + jax.lax.dot_general(
        w.astype(jnp.bfloat16), x.astype(jnp.bfloat16),
        (((1,), (0,)), ((), ())), preferred_element_type=jnp.float32)
    m_ref[...] = m_new

    @pl.when(i == NB - 1)
    def _fin():
        pooled = acc_ref[...] / (s_ref[...] + 1e-16)
        out_ref[...] = jax.lax.dot_general(
            pooled, wv_ref[...], (((1,), (1,)), ((), ())),
            preferred_element_type=jnp.float32,
            precision=jax.lax.Precision.HIGHEST)


def kernel(x, batch, query, Wk, Wv):
    b3 = batch.reshape(NB, 1, B)
    q2 = query.reshape(1, D)
    return pl.pallas_call(
        _body,
        grid=(NB,),
        in_specs=[
            pl.BlockSpec((1, 1, B), lambda i: (i, 0, 0)),
            pl.BlockSpec((B, D), lambda i: (i, 0)),
            pl.BlockSpec((1, D), lambda i: (0, 0)),
            pl.BlockSpec((D, D), lambda i: (0, 0)),
            pl.BlockSpec((D, D), lambda i: (0, 0)),
        ],
        out_specs=pl.BlockSpec((G, D), lambda i: (0, 0)),
        out_shape=jax.ShapeDtypeStruct((G, D), jnp.float32),
        scratch_shapes=[
            pltpu.VMEM((G, 1), jnp.float32),
            pltpu.VMEM((G, 1), jnp.float32),
            pltpu.VMEM((G, D), jnp.float32),
        ],
    )(b3, x, q2, Wk, Wv)
